# baseline (speedup 1.0000x reference)
import functools
import jax
import jax.numpy as jnp
from jax import lax
from jax.experimental import pallas as pl
from jax.experimental.pallas import tpu as pltpu
from jax.experimental.pallas import tpu_sc as plsc

SEQ = 4096
DM = 1024
BATCH = 4
NW = 32
ROWS_PER_W = SEQ // NW
CH = 64


def _sc_body(table_hbm, out_hbm, buf, sem):
    w = lax.axis_index("s") * 2 + lax.axis_index("c")
    base = w * ROWS_PER_W
    cps = [pltpu.make_async_copy(table_hbm.at[pl.ds(base, ROWS_PER_W), :],
                                 out_hbm.at[b, pl.ds(base, ROWS_PER_W), :], sem)
           for b in range(BATCH)]
    for c in cps:
        c.start()
    for c in cps:
        c.wait()


def kernel(input_ids, pos_table):
    mesh = plsc.VectorSubcoreMesh(core_axis_name="c", subcore_axis_name="s")
    k = functools.partial(
        pl.kernel,
        mesh=mesh,
        out_type=jax.ShapeDtypeStruct((BATCH, SEQ, DM), pos_table.dtype),
        scratch_types=[
            pltpu.VMEM((CH, DM), pos_table.dtype),
            pltpu.SemaphoreType.DMA,
        ],
    )(_sc_body)
    return k(pos_table)


# speedup vs baseline: 45.0594x; 45.0594x over previous
import functools
import jax
import jax.numpy as jnp
from jax import lax
from jax.experimental import pallas as pl
from jax.experimental.pallas import tpu as pltpu
from jax.experimental.pallas import tpu_sc as plsc

SEQ = 4096
DM = 1024
BATCH = 4
NW = 32
ROWS_PER_W = SEQ // NW
CH = 32


def _sc_body(table_hbm, out_hbm, buf, isem, osem):
    w = lax.axis_index("s") * 2 + lax.axis_index("c")
    base = w * ROWS_PER_W
    nch = ROWS_PER_W // CH

    def inc(ci):
        return pltpu.make_async_copy(
            table_hbm.at[pl.ds(base + ci * CH, CH), :], buf.at[ci % 2], isem)

    def outc(ci, b):
        return pltpu.make_async_copy(
            buf.at[ci % 2], out_hbm.at[b, pl.ds(base + ci * CH, CH), :], osem)

    inc(0).start()
    for ci in range(nch):
        if ci + 1 < nch:
            if ci >= 1:
                for b in range(BATCH):
                    outc(ci - 1, b).wait()
            inc(ci + 1).start()
        inc(ci).wait()
        for b in range(BATCH):
            outc(ci, b).start()
    for ci in range(max(nch - 2, 0), nch):
        for b in range(BATCH):
            outc(ci, b).wait()


def kernel(input_ids, pos_table):
    mesh = plsc.VectorSubcoreMesh(core_axis_name="c", subcore_axis_name="s")
    k = functools.partial(
        pl.kernel,
        mesh=mesh,
        out_type=jax.ShapeDtypeStruct((BATCH, SEQ, DM), pos_table.dtype),
        scratch_types=[
            pltpu.VMEM((2, CH, DM), pos_table.dtype),
            pltpu.SemaphoreType.DMA,
            pltpu.SemaphoreType.DMA,
        ],
    )(_sc_body)
    return k(pos_table)


# speedup vs baseline: 82.0670x; 1.8213x over previous
import jax
import jax.numpy as jnp
from jax.experimental import pallas as pl
from jax.experimental.pallas import tpu as pltpu

CHUNK_ROWS = (256, 256, 512, 1024, 1024, 1024)


def _copy_body(table_hbm, out_hbm, buf, in_sems, out_sem):
    batch = out_hbm.shape[0]
    offs = [0]
    for c in CHUNK_ROWS:
        offs.append(offs[-1] + c)

    def in_copy(i):
        return pltpu.make_async_copy(
            table_hbm.at[pl.ds(offs[i], CHUNK_ROWS[i]), :],
            buf.at[pl.ds(offs[i], CHUNK_ROWS[i]), :], in_sems.at[i])

    def out_copy(i, b):
        return pltpu.make_async_copy(
            buf.at[pl.ds(offs[i], CHUNK_ROWS[i]), :],
            out_hbm.at[b, pl.ds(offs[i], CHUNK_ROWS[i]), :], out_sem)

    n = len(CHUNK_ROWS)
    for i in range(n):
        in_copy(i).start()
    for i in range(n):
        in_copy(i).wait()
        for b in range(batch):
            out_copy(i, b).start()
    for i in range(n):
        for b in range(batch):
            out_copy(i, b).wait()


def kernel(input_ids, pos_table):
    batch, seq_len = input_ids.shape
    d_model = pos_table.shape[1]
    out = pl.pallas_call(
        _copy_body,
        in_specs=[pl.BlockSpec(memory_space=pl.ANY)],
        out_specs=pl.BlockSpec(memory_space=pl.ANY),
        out_shape=jax.ShapeDtypeStruct((batch, seq_len, d_model), pos_table.dtype),
        scratch_shapes=[
            pltpu.VMEM((seq_len, d_model), pos_table.dtype),
            pltpu.SemaphoreType.DMA((len(CHUNK_ROWS),)),
            pltpu.SemaphoreType.DMA,
        ],
    )(pos_table)
    return out
